# baseline (speedup 1.0000x reference)
import functools

import jax
import jax.numpy as jnp
from jax import lax
from jax.experimental import pallas as pl
from jax.experimental.pallas import tpu as pltpu
from jax.experimental.pallas import tpu_sc as plsc

H = 128
NW = 32
LANES = 16


def _edge_diffprod_kernel(E, EW, CH):
    n_chunks = EW // CH
    n_pairs = n_chunks // 2
    assert n_chunks == 2 * n_pairs + 1

    mesh = plsc.VectorSubcoreMesh(core_axis_name="c", subcore_axis_name="s")

    @functools.partial(
        pl.kernel,
        out_type=jax.ShapeDtypeStruct((E, H // 2), jnp.int32),
        mesh=mesh,
        scratch_types=[
            pltpu.VMEM((EW,), jnp.int32),
            pltpu.VMEM((EW,), jnp.int32),
            pltpu.VMEM((2, CH, H), jnp.int32),
            pltpu.VMEM((2, CH, H), jnp.int32),
            pltpu.VMEM((2, CH, H // 2), jnp.int32),
            pltpu.SemaphoreType.DMA,
            pltpu.SemaphoreType.DMA,
            pltpu.SemaphoreType.DMA,
            pltpu.SemaphoreType.DMA,
            pltpu.SemaphoreType.DMA,
            pltpu.SemaphoreType.DMA,
        ],
    )
    def edge_kernel(t_hbm, k_hbm, i_hbm, h_hbm, kidx_all, iidx_all,
                    krows, irows, hbuf, sk0, sk1, si0, si1, st0, st1):
        wid = lax.axis_index("s") * 2 + lax.axis_index("c")
        base = wid * EW
        pltpu.sync_copy(k_hbm.at[pl.ds(base, EW)], kidx_all)
        pltpu.sync_copy(i_hbm.at[pl.ds(base, EW)], iidx_all)
        semk = (sk0, sk1)
        semi = (si0, si1)
        semst = (st0, st1)

        def fire(c, b):
            pltpu.async_copy(
                t_hbm.at[kidx_all.at[pl.ds(c * CH, CH)]], krows.at[b],
                semk[b])
            pltpu.async_copy(
                t_hbm.at[iidx_all.at[pl.ds(c * CH, CH)]], irows.at[b],
                semi[b])

        def wait_gather(b):
            pltpu.make_async_copy(
                t_hbm.at[kidx_all.at[pl.ds(0, CH)]], krows.at[b],
                semk[b]).wait()
            pltpu.make_async_copy(
                t_hbm.at[iidx_all.at[pl.ds(0, CH)]], irows.at[b],
                semi[b]).wait()

        MASK_HI = jnp.int32(-65536)
        MASK_LO = jnp.int32(65535)

        def _hi(wv):
            return lax.bitcast_convert_type(wv & MASK_HI, jnp.float32)

        def _lo(wv):
            return lax.bitcast_convert_type(wv << 16, jnp.float32)

        def compute(b):
            def row_body(e, carry2):
                for l in range(H // (2 * LANES)):
                    o = l * LANES
                    ku = krows[b, e, pl.ds(o, LANES)]
                    iu = irows[b, e, pl.ds(o, LANES)]
                    kx = krows[b, e, pl.ds(H // 2 + o, LANES)]
                    ix = irows[b, e, pl.ds(H // 2 + o, LANES)]
                    h_h = lax.bitcast_convert_type(
                        (_hi(ku) - _hi(iu)) * (_hi(kx) - _hi(ix)), jnp.int32)
                    h_l = lax.bitcast_convert_type(
                        (_lo(ku) - _lo(iu)) * (_lo(kx) - _lo(ix)), jnp.int32)
                    hbuf[b, e, pl.ds(o, LANES)] = \
                        (h_h & MASK_HI) | ((h_l >> 16) & MASK_LO)
                return carry2

            lax.fori_loop(0, CH, row_body, 0)

        def fire_store(c, b):
            pltpu.async_copy(hbuf.at[b],
                             h_hbm.at[pl.ds(base + c * CH, CH)], semst[b])

        def wait_store(b):
            pltpu.make_async_copy(hbuf.at[b], h_hbm.at[pl.ds(base, CH)],
                                  semst[b]).wait()

        fire(0, 0)
        fire(1, 1)

        def pair_body(p, carry):
            c0 = 2 * p
            wait_gather(0)

            @pl.when(p > 0)
            def _():
                wait_store(0)

            compute(0)
            fire_store(c0, 0)
            fire(c0 + 2, 0)
            wait_gather(1)

            @pl.when(p > 0)
            def _():
                wait_store(1)

            compute(1)
            fire_store(c0 + 1, 1)

            @pl.when(p < n_pairs - 1)
            def _():
                fire(c0 + 3, 1)

            return carry

        lax.fori_loop(0, n_pairs, pair_body, 0)

        wait_gather(0)
        wait_store(0)
        compute(0)
        pltpu.sync_copy(hbuf.at[0],
                        h_hbm.at[pl.ds(base + (n_chunks - 1) * CH, CH)])
        wait_store(1)

    return edge_kernel


def _logits_call(h0, w1t, b1, w2, E, EB):
    def body(h_ref, w1t_ref, b1_ref, w2_ref, out_ref):
        h = jnp.dot(h_ref[...], w1t_ref[...],
                    preferred_element_type=jnp.float32)
        h = jnp.maximum(h + b1_ref[...], 0.0)
        out_ref[...] = lax.dot_general(
            w2_ref[...], h, (((1,), (1,)), ((), ())),
            preferred_element_type=jnp.float32).reshape(1, 1, EB)

    return pl.pallas_call(
        body,
        grid=(E // EB,),
        in_specs=[
            pl.BlockSpec((EB, H), lambda b: (b, 0)),
            pl.BlockSpec((H, H), lambda b: (0, 0)),
            pl.BlockSpec((1, H), lambda b: (0, 0)),
            pl.BlockSpec((1, H), lambda b: (0, 0)),
        ],
        out_specs=pl.BlockSpec((1, 1, EB), lambda b: (b, 0, 0)),
        out_shape=jax.ShapeDtypeStruct((E // EB, 1, EB), jnp.float32),
    )(h0, w1t, b1, w2)


def _softmax_call(logits2d):
    def body(l_ref, out_ref):
        l = l_ref[...]
        m = jnp.max(l)
        e = jnp.exp(l - m)
        out_ref[...] = e / jnp.sum(e)

    return pl.pallas_call(
        body,
        out_shape=jax.ShapeDtypeStruct(logits2d.shape, jnp.float32),
    )(logits2d)


def _neighbor_agg_kernel(N, D, CN):
    PAIRS = CN * D
    nodes_w = (N // NW) // CN * CN
    n_chunks = nodes_w // CN
    n_pairs = n_chunks // 2
    assert n_chunks == 2 * n_pairs
    rem = N - NW * nodes_w
    assert rem <= NW
    jpre = n_chunks * PAIRS

    mesh = plsc.VectorSubcoreMesh(core_axis_name="c", subcore_axis_name="s")

    @functools.partial(
        pl.kernel,
        out_type=jax.ShapeDtypeStruct((N, H), jnp.float32),
        mesh=mesh,
        scratch_types=[
            pltpu.VMEM((jpre,), jnp.int32),
            pltpu.VMEM((2, PAIRS), jnp.int32),
            pltpu.VMEM((PAIRS + LANES,), jnp.float32),
            pltpu.VMEM((PAIRS + LANES,), jnp.float32),
            pltpu.VMEM((2, PAIRS, H), jnp.float32),
            pltpu.VMEM((nodes_w, H), jnp.float32),
            pltpu.VMEM((D,), jnp.int32),
            pltpu.VMEM((D,), jnp.int32),
            pltpu.VMEM((D + LANES,), jnp.float32),
            pltpu.VMEM((D, H), jnp.float32),
            pltpu.SemaphoreType.DMA,
            pltpu.SemaphoreType.DMA,
            pltpu.SemaphoreType.DMA,
            pltpu.SemaphoreType.DMA,
            pltpu.SemaphoreType.DMA,
            pltpu.SemaphoreType.DMA,
        ],
    )
    def agg_kernel(naf_hbm, kpad_hbm, apad_hbm, x_hbm, agg_hbm,
                   jidx_all, kj, av0, av1, rows, outall, ej, ekj, eav, erows,
                   ska, skb, saa, sab, sra, srb):
        av = (av0, av1)
        wid = lax.axis_index("s") * 2 + lax.axis_index("c")
        node0 = wid * nodes_w
        pltpu.sync_copy(naf_hbm.at[pl.ds(node0 * D, jpre)], jidx_all)
        semk = (ska, skb)
        sema = (saa, sab)
        semr = (sra, srb)

        def fire_kjav(c, b):
            idx = jidx_all.at[pl.ds(c * PAIRS, PAIRS)]
            pltpu.async_copy(kpad_hbm.at[idx], kj.at[b], semk[b])
            pltpu.async_copy(apad_hbm.at[idx],
                             av[b].at[pl.ds(0, PAIRS)], sema[b])

        def wait_kjav(b):
            idx = jidx_all.at[pl.ds(0, PAIRS)]
            pltpu.make_async_copy(kpad_hbm.at[idx], kj.at[b], semk[b]).wait()
            pltpu.make_async_copy(apad_hbm.at[idx],
                                  av[b].at[pl.ds(0, PAIRS)],
                                  sema[b]).wait()

        def fire_rows(b):
            pltpu.async_copy(x_hbm.at[kj.at[b]], rows.at[b], semr[b])

        def wait_rows(b):
            pltpu.make_async_copy(x_hbm.at[kj.at[b]], rows.at[b],
                                  semr[b]).wait()

        def compute(c, b):
            for n in range(CN):
                def d_body(d, acc):
                    cidx = n * D + d
                    a = av[b][pl.ds(cidx, LANES)][0]
                    return tuple(
                        acc[l] + a * rows[b, cidx, pl.ds(l * LANES, LANES)]
                        for l in range(H // LANES))

                zero = jnp.zeros((LANES,), jnp.float32)
                acc = lax.fori_loop(0, D, d_body,
                                    tuple(zero for _ in range(H // LANES)))
                row = c * CN + n
                for l in range(H // LANES):
                    outall[row, pl.ds(l * LANES, LANES)] = acc[l]

        fire_kjav(0, 0)
        wait_kjav(0)
        fire_rows(0)
        fire_kjav(1, 1)

        def pair_body(p, carry):
            c0 = 2 * p
            wait_kjav(1)
            fire_rows(1)
            wait_rows(0)
            compute(c0, 0)

            @pl.when(p < n_pairs - 1)
            def _():
                fire_kjav(c0 + 2, 0)

            wait_rows(1)
            compute(c0 + 1, 1)

            @pl.when(p < n_pairs - 1)
            def _():
                wait_kjav(0)
                fire_rows(0)
                fire_kjav(c0 + 3, 1)

            return carry

        lax.fori_loop(0, n_pairs, pair_body, 0)
        pltpu.sync_copy(outall, agg_hbm.at[pl.ds(node0, nodes_w)])

        @pl.when(wid < rem)
        def _():
            g = NW * nodes_w + wid
            pltpu.sync_copy(naf_hbm.at[pl.ds(g * D, D)], ej)
            cpk = pltpu.async_copy(kpad_hbm.at[ej], ekj, ska)
            cpa = pltpu.async_copy(apad_hbm.at[ej], eav.at[pl.ds(0, D)], saa)
            cpk.wait()
            cpa.wait()
            cpr = pltpu.async_copy(x_hbm.at[ekj], erows, sra)
            cpr.wait()

            def d_body(d, acc):
                a = eav[pl.ds(d, LANES)][0]
                return tuple(
                    acc[l] + a * erows[d, pl.ds(l * LANES, LANES)]
                    for l in range(H // LANES))

            zero = jnp.zeros((LANES,), jnp.float32)
            acc = lax.fori_loop(0, D, d_body,
                                tuple(zero for _ in range(H // LANES)))
            for l in range(H // LANES):
                erows[0, pl.ds(l * LANES, LANES)] = acc[l]
            pltpu.sync_copy(erows.at[pl.ds(0, 1)], agg_hbm.at[pl.ds(g, 1)])

    return agg_kernel


def _head_call(x, agg, w0, w1, fc1t, fc1_b, fc2t, fc2_b, N, NB, OUT):
    def body(x_ref, agg_ref, w0_ref, w1_ref, fc1t_ref, fc1b_ref, fc2t_ref,
             fc2b_ref, out_ref):
        x2 = (jnp.dot(x_ref[...], w0_ref[...],
                      preferred_element_type=jnp.float32)
              + jnp.dot(agg_ref[...], w1_ref[...],
                        preferred_element_type=jnp.float32))
        x2 = jnp.maximum(jnp.dot(x2, fc1t_ref[...],
                                 preferred_element_type=jnp.float32)
                         + fc1b_ref[...], 0.0)
        out_ref[...] = jnp.dot(x2, fc2t_ref[...],
                               preferred_element_type=jnp.float32) \
            + fc2b_ref[...]

    return pl.pallas_call(
        body,
        grid=(N // NB,),
        in_specs=[
            pl.BlockSpec((NB, H), lambda b: (b, 0)),
            pl.BlockSpec((NB, H), lambda b: (b, 0)),
            pl.BlockSpec((H, H), lambda b: (0, 0)),
            pl.BlockSpec((H, H), lambda b: (0, 0)),
            pl.BlockSpec((H, H), lambda b: (0, 0)),
            pl.BlockSpec((1, H), lambda b: (0, 0)),
            pl.BlockSpec((H, OUT), lambda b: (0, 0)),
            pl.BlockSpec((1, OUT), lambda b: (0, 0)),
        ],
        out_specs=pl.BlockSpec((NB, OUT), lambda b: (b, 0)),
        out_shape=jax.ShapeDtypeStruct((N, OUT), jnp.float32),
    )(x, agg, w0, w1, fc1t, fc1_b, fc2t, fc2_b)


def kernel(u, edge_index, neighbor_all, emb_id, att_fc1_w, att_fc1_b,
           att_fc2_w, att_fc2_b, w, fc1_w, fc1_b, fc2_w, fc2_b):
    N, Hdim = u.shape
    E = edge_index.shape[1]
    D = neighbor_all.shape[1]
    OUT = fc2_w.shape[0]
    assert Hdim == H

    x = emb_id
    k = edge_index[0]
    i = edge_index[1]

    t_bf = jnp.concatenate([u, x], axis=1).astype(jnp.bfloat16)
    t32 = lax.bitcast_convert_type(t_bf.reshape(N, H, 2), jnp.int32)
    EW = E // NW
    CH = 80
    h032 = _edge_diffprod_kernel(E, EW, CH)(t32, k, i)
    h0 = lax.bitcast_convert_type(h032, jnp.bfloat16).reshape(E, H)

    EB = 512
    logits = _logits_call(h0, att_fc1_w.T.astype(jnp.bfloat16),
                          att_fc1_b.reshape(1, H),
                          att_fc2_w, E, EB).reshape(E // EB, EB)
    alpha2d = _softmax_call(logits)

    PAD = 8
    alpha_pad = jnp.concatenate(
        [alpha2d.reshape(E), jnp.zeros((PAD,), jnp.float32)])
    k_pad = jnp.concatenate([k, jnp.zeros((PAD,), jnp.int32)])
    naf = neighbor_all.reshape(N * D)
    CN = 128 // D
    agg = _neighbor_agg_kernel(N, D, CN)(naf, k_pad, alpha_pad, x)

    NB = 1000
    return _head_call(x, agg, w[0], w[1], fc1_w.T, fc1_b.reshape(1, H),
                      fc2_w.T, fc2_b.reshape(1, OUT), N, NB, OUT)


# speedup vs baseline: 2.2281x; 2.2281x over previous
import functools

import jax
import jax.numpy as jnp
from jax import lax
from jax.experimental import pallas as pl
from jax.experimental.pallas import tpu as pltpu
from jax.experimental.pallas import tpu_sc as plsc

H = 128
NW = 32
LANES = 16


def _edge_diffprod_kernel(E, EW, CH):
    n_chunks = EW // CH
    n_pairs = n_chunks // 2
    assert n_chunks == 2 * n_pairs + 1

    mesh = plsc.VectorSubcoreMesh(core_axis_name="c", subcore_axis_name="s")

    @functools.partial(
        pl.kernel,
        out_type=jax.ShapeDtypeStruct((E, H // 2), jnp.int32),
        mesh=mesh,
        scratch_types=[
            pltpu.VMEM((EW,), jnp.int32),
            pltpu.VMEM((EW,), jnp.int32),
            pltpu.VMEM((2, CH, H), jnp.int32),
            pltpu.VMEM((2, CH, H), jnp.int32),
            pltpu.VMEM((2, CH, H // 2), jnp.int32),
            pltpu.SemaphoreType.DMA,
            pltpu.SemaphoreType.DMA,
            pltpu.SemaphoreType.DMA,
            pltpu.SemaphoreType.DMA,
            pltpu.SemaphoreType.DMA,
            pltpu.SemaphoreType.DMA,
        ],
    )
    def edge_kernel(t_hbm, k_hbm, i_hbm, h_hbm, kidx_all, iidx_all,
                    krows, irows, hbuf, sk0, sk1, si0, si1, st0, st1):
        wid = lax.axis_index("s") * 2 + lax.axis_index("c")
        base = wid * EW
        pltpu.sync_copy(k_hbm.at[pl.ds(base, EW)], kidx_all)
        pltpu.sync_copy(i_hbm.at[pl.ds(base, EW)], iidx_all)
        semk = (sk0, sk1)
        semi = (si0, si1)
        semst = (st0, st1)

        def fire(c, b):
            pltpu.async_copy(
                t_hbm.at[kidx_all.at[pl.ds(c * CH, CH)]], krows.at[b],
                semk[b])
            pltpu.async_copy(
                t_hbm.at[iidx_all.at[pl.ds(c * CH, CH)]], irows.at[b],
                semi[b])

        def wait_gather(b):
            pltpu.make_async_copy(
                t_hbm.at[kidx_all.at[pl.ds(0, CH)]], krows.at[b],
                semk[b]).wait()
            pltpu.make_async_copy(
                t_hbm.at[iidx_all.at[pl.ds(0, CH)]], irows.at[b],
                semi[b]).wait()

        MASK_HI = jnp.int32(-65536)
        MASK_LO = jnp.int32(65535)

        def _hi(wv):
            return lax.bitcast_convert_type(wv, jnp.float32)

        def _lo(wv):
            return lax.bitcast_convert_type(wv << 16, jnp.float32)

        def compute(b):
            def row_body(e, carry2):
                for l in range(H // (2 * LANES)):
                    o = l * LANES
                    ku = krows[b, e, pl.ds(o, LANES)]
                    iu = irows[b, e, pl.ds(o, LANES)]
                    kx = krows[b, e, pl.ds(H // 2 + o, LANES)]
                    ix = irows[b, e, pl.ds(H // 2 + o, LANES)]
                    h_h = lax.bitcast_convert_type(
                        (_hi(ku) - _hi(iu)) * (_hi(kx) - _hi(ix)), jnp.int32)
                    h_l = lax.bitcast_convert_type(
                        (_lo(ku) - _lo(iu)) * (_lo(kx) - _lo(ix)), jnp.int32)
                    hbuf[b, e, pl.ds(o, LANES)] = \
                        (h_h & MASK_HI) | ((h_l >> 16) & MASK_LO)
                return carry2

            lax.fori_loop(0, CH, row_body, 0)

        def fire_store(c, b):
            pltpu.async_copy(hbuf.at[b],
                             h_hbm.at[pl.ds(base + c * CH, CH)], semst[b])

        def wait_store(b):
            pltpu.make_async_copy(hbuf.at[b], h_hbm.at[pl.ds(base, CH)],
                                  semst[b]).wait()

        fire(0, 0)
        fire(1, 1)

        def pair_body(p, carry):
            c0 = 2 * p
            wait_gather(0)

            @pl.when(p > 0)
            def _():
                wait_store(0)

            compute(0)
            fire_store(c0, 0)
            fire(c0 + 2, 0)
            wait_gather(1)

            @pl.when(p > 0)
            def _():
                wait_store(1)

            compute(1)
            fire_store(c0 + 1, 1)

            @pl.when(p < n_pairs - 1)
            def _():
                fire(c0 + 3, 1)

            return carry

        lax.fori_loop(0, n_pairs, pair_body, 0)

        wait_gather(0)
        wait_store(0)
        compute(0)
        pltpu.sync_copy(hbuf.at[0],
                        h_hbm.at[pl.ds(base + (n_chunks - 1) * CH, CH)])
        wait_store(1)

    return edge_kernel


MASK16 = 65535
MASKHI = -65536


def _pack_half(v):
    r_lo = lax.bitcast_convert_type(
        v[:, 0:H // 2].astype(jnp.bfloat16).astype(jnp.float32), jnp.int32)
    r_hi = lax.bitcast_convert_type(
        v[:, H // 2:H].astype(jnp.bfloat16).astype(jnp.float32), jnp.int32)
    return ((r_lo >> 16) & MASK16) | (r_hi & MASKHI)


def _unpack_cat(w):
    f_lo = lax.bitcast_convert_type(w << 16, jnp.float32)
    f_hi = lax.bitcast_convert_type(w & MASKHI, jnp.float32)
    return jnp.concatenate([f_lo, f_hi], axis=1)


def _pack_table_call(u, x, N):
    def body(u_ref, x_ref, t_ref):
        t_ref[...] = jnp.concatenate(
            [_pack_half(u_ref[...]), _pack_half(x_ref[...])], axis=1)

    NB = 2000
    return pl.pallas_call(
        body,
        grid=(N // NB,),
        in_specs=[
            pl.BlockSpec((NB, H), lambda b: (b, 0)),
            pl.BlockSpec((NB, H), lambda b: (b, 0)),
        ],
        out_specs=pl.BlockSpec((NB, H), lambda b: (b, 0)),
        out_shape=jax.ShapeDtypeStruct((N, H), jnp.int32),
    )(u, x)


def _logits_call(h032, w1t, b1, w2, E, EB):
    def body(h_ref, w1t_ref, b1_ref, w2_ref, out_ref):
        hbf = _unpack_cat(h_ref[...]).astype(jnp.bfloat16)
        h = jnp.dot(hbf, w1t_ref[...],
                    preferred_element_type=jnp.float32)
        h = jnp.maximum(h + b1_ref[...], 0.0)
        out_ref[...] = lax.dot_general(
            w2_ref[...], h, (((1,), (1,)), ((), ())),
            preferred_element_type=jnp.float32).reshape(1, 1, EB)

    return pl.pallas_call(
        body,
        grid=(E // EB,),
        in_specs=[
            pl.BlockSpec((EB, H // 2), lambda b: (b, 0)),
            pl.BlockSpec((H, H), lambda b: (0, 0)),
            pl.BlockSpec((1, H), lambda b: (0, 0)),
            pl.BlockSpec((1, H), lambda b: (0, 0)),
        ],
        out_specs=pl.BlockSpec((1, 1, EB), lambda b: (b, 0, 0)),
        out_shape=jax.ShapeDtypeStruct((E // EB, 1, EB), jnp.float32),
    )(h032, w1t, b1, w2)


def _neighbor_agg_kernel(N, D, CN, E):
    PAIRS = CN * D
    nodes_w = (N // NW) // CN * CN
    n_chunks = nodes_w // CN
    n_pairs = n_chunks // 2
    assert n_chunks == 2 * n_pairs
    rem = N - NW * nodes_w
    assert rem <= NW
    jpre = n_chunks * PAIRS
    ES = E // (NW // 2)

    mesh = plsc.VectorSubcoreMesh(core_axis_name="c", subcore_axis_name="s")

    @functools.partial(
        pl.kernel,
        out_type=jax.ShapeDtypeStruct((N, H), jnp.float32),
        mesh=mesh,
        scratch_types=[
            pltpu.VMEM((jpre,), jnp.int32),
            pltpu.VMEM((2, PAIRS), jnp.int32),
            pltpu.VMEM((PAIRS + LANES,), jnp.float32),
            pltpu.VMEM((PAIRS + LANES,), jnp.float32),
            pltpu.VMEM((2, PAIRS, H), jnp.float32),
            pltpu.VMEM((nodes_w, H), jnp.float32),
            pltpu.VMEM((D,), jnp.int32),
            pltpu.VMEM((D,), jnp.int32),
            pltpu.VMEM((D + LANES,), jnp.float32),
            pltpu.VMEM((D, H), jnp.float32),
            pltpu.VMEM((8, H), jnp.float32),
            pltpu.VMEM((ES,), jnp.float32),
            pltpu.VMEM((LANES,), jnp.float32),
            pltpu.VMEM((LANES, LANES), jnp.float32),
            pltpu.VMEM_SHARED((2 * LANES, LANES), jnp.float32),
            pltpu.SemaphoreType.DMA,
            pltpu.SemaphoreType.DMA,
            pltpu.SemaphoreType.DMA,
            pltpu.SemaphoreType.DMA,
            pltpu.SemaphoreType.DMA,
            pltpu.SemaphoreType.DMA,
        ],
    )
    def agg_kernel(naf_hbm, kpad_hbm, lg_hbm, x_hbm, agg_hbm,
                   jidx_all, kj, av0, av1, rows, outall, ej, ekj, eav, erows,
                   eout, lgbuf, pvec, locbuf, shpart,
                   ska, skb, saa, sab, sra, srb):
        av = (av0, av1)
        wid = lax.axis_index("s") * 2 + lax.axis_index("c")
        sid = lax.axis_index("s")
        node0 = wid * nodes_w
        pltpu.sync_copy(naf_hbm.at[pl.ds(node0 * D, jpre)], jidx_all)
        semk = (ska, skb)
        sema = (saa, sab)
        semr = (sra, srb)

        pltpu.sync_copy(lg_hbm.at[pl.ds(sid * ES, ES)], lgbuf)

        def max_body(l, m):
            return jnp.maximum(m, lgbuf[pl.ds(l * LANES, LANES)])

        pm = lax.fori_loop(0, ES // LANES, max_body,
                           jnp.full((LANES,), -jnp.inf, jnp.float32))
        pvec[...] = pm
        pltpu.sync_copy(pvec, shpart.at[sid])
        plsc.subcore_barrier()
        pltpu.sync_copy(shpart.at[pl.ds(0, LANES)], locbuf)
        gm = locbuf[0, pl.ds(0, LANES)]
        for w in range(1, LANES):
            gm = jnp.maximum(gm, locbuf[w, pl.ds(0, LANES)])
        M = gm[0]
        for i in range(1, LANES):
            M = jnp.maximum(M, gm[i])

        def sum_body(l, s):
            return s + jnp.exp(lgbuf[pl.ds(l * LANES, LANES)] - M)

        ps = lax.fori_loop(0, ES // LANES, sum_body,
                           jnp.zeros((LANES,), jnp.float32))
        pvec[...] = ps
        pltpu.sync_copy(pvec, shpart.at[LANES + sid])
        plsc.subcore_barrier()
        pltpu.sync_copy(shpart.at[pl.ds(LANES, LANES)], locbuf)
        gs = locbuf[0, pl.ds(0, LANES)]
        for w in range(1, LANES):
            gs = gs + locbuf[w, pl.ds(0, LANES)]
        S = gs[0]
        for i in range(1, LANES):
            S = S + gs[i]
        rinv = jnp.ones((LANES,), jnp.float32) / jnp.full((LANES,), S,
                                                          jnp.float32)

        def xform(b):
            for j in range(PAIRS // LANES):
                sl = pl.ds(j * LANES, LANES)
                av[b][sl] = jnp.exp(av[b][sl] - M) * rinv

        def fire_kjav(c, b):
            idx = jidx_all.at[pl.ds(c * PAIRS, PAIRS)]
            pltpu.async_copy(kpad_hbm.at[idx], kj.at[b], semk[b])
            pltpu.async_copy(lg_hbm.at[idx],
                             av[b].at[pl.ds(0, PAIRS)], sema[b])

        def wait_kjav(b):
            idx = jidx_all.at[pl.ds(0, PAIRS)]
            pltpu.make_async_copy(kpad_hbm.at[idx], kj.at[b], semk[b]).wait()
            pltpu.make_async_copy(lg_hbm.at[idx],
                                  av[b].at[pl.ds(0, PAIRS)],
                                  sema[b]).wait()

        def fire_rows(b):
            pltpu.async_copy(x_hbm.at[kj.at[b]], rows.at[b], semr[b])

        def wait_rows(b):
            pltpu.make_async_copy(x_hbm.at[kj.at[b]], rows.at[b],
                                  semr[b]).wait()

        def compute(c, b):
            for n in range(CN):
                def d_body(d, acc):
                    cidx = n * D + d
                    a = av[b][pl.ds(cidx, LANES)][0]
                    return tuple(
                        acc[l] + a * rows[b, cidx, pl.ds(l * LANES, LANES)]
                        for l in range(H // LANES))

                zero = jnp.zeros((LANES,), jnp.float32)
                acc = lax.fori_loop(0, D, d_body,
                                    tuple(zero for _ in range(H // LANES)))
                row = c * CN + n
                for l in range(H // LANES):
                    outall[row, pl.ds(l * LANES, LANES)] = acc[l]

        fire_kjav(0, 0)
        wait_kjav(0)
        xform(0)
        fire_rows(0)
        fire_kjav(1, 1)

        def pair_body(p, carry):
            c0 = 2 * p
            wait_kjav(1)
            xform(1)
            fire_rows(1)
            wait_rows(0)
            compute(c0, 0)

            @pl.when(p < n_pairs - 1)
            def _():
                fire_kjav(c0 + 2, 0)

            wait_rows(1)
            compute(c0 + 1, 1)

            @pl.when(p < n_pairs - 1)
            def _():
                wait_kjav(0)
                xform(0)
                fire_rows(0)
                fire_kjav(c0 + 3, 1)

            return carry

        lax.fori_loop(0, n_pairs, pair_body, 0)
        pltpu.sync_copy(outall, agg_hbm.at[pl.ds(node0, nodes_w)])

        @pl.when(wid < rem)
        def _():
            g = NW * nodes_w + wid
            pltpu.sync_copy(naf_hbm.at[pl.ds(g * D, D)], ej)
            cpk = pltpu.async_copy(kpad_hbm.at[ej], ekj, ska)
            cpa = pltpu.async_copy(lg_hbm.at[ej], eav.at[pl.ds(0, D)], saa)
            cpk.wait()
            cpa.wait()
            for j in range(D // LANES):
                sl = pl.ds(j * LANES, LANES)
                eav[sl] = jnp.exp(eav[sl] - M) * rinv
            cpr = pltpu.async_copy(x_hbm.at[ekj], erows, sra)
            cpr.wait()

            def d_body(d, acc):
                a = eav[pl.ds(d, LANES)][0]
                return tuple(
                    acc[l] + a * erows[d, pl.ds(l * LANES, LANES)]
                    for l in range(H // LANES))

            zero = jnp.zeros((LANES,), jnp.float32)
            acc = lax.fori_loop(0, D, d_body,
                                tuple(zero for _ in range(H // LANES)))
            for l in range(H // LANES):
                eout[0, pl.ds(l * LANES, LANES)] = acc[l]
            pltpu.sync_copy(eout.at[pl.ds(0, 1)], agg_hbm.at[pl.ds(g, 1)])

    return agg_kernel


def _head_call(x, agg, w0, w1, fc1t, fc1_b, fc2t, fc2_b, N, NB, OUT):
    def body(x_ref, agg_ref, w0_ref, w1_ref, fc1t_ref, fc1b_ref, fc2t_ref,
             fc2b_ref, out_ref):
        x2 = (jnp.dot(x_ref[...], w0_ref[...],
                      preferred_element_type=jnp.float32)
              + jnp.dot(agg_ref[...], w1_ref[...],
                        preferred_element_type=jnp.float32))
        x2 = jnp.maximum(jnp.dot(x2, fc1t_ref[...],
                                 preferred_element_type=jnp.float32)
                         + fc1b_ref[...], 0.0)
        out_ref[...] = jnp.dot(x2, fc2t_ref[...],
                               preferred_element_type=jnp.float32) \
            + fc2b_ref[...]

    return pl.pallas_call(
        body,
        grid=(N // NB,),
        in_specs=[
            pl.BlockSpec((NB, H), lambda b: (b, 0)),
            pl.BlockSpec((NB, H), lambda b: (b, 0)),
            pl.BlockSpec((H, H), lambda b: (0, 0)),
            pl.BlockSpec((H, H), lambda b: (0, 0)),
            pl.BlockSpec((H, H), lambda b: (0, 0)),
            pl.BlockSpec((1, H), lambda b: (0, 0)),
            pl.BlockSpec((H, OUT), lambda b: (0, 0)),
            pl.BlockSpec((1, OUT), lambda b: (0, 0)),
        ],
        out_specs=pl.BlockSpec((NB, OUT), lambda b: (b, 0)),
        out_shape=jax.ShapeDtypeStruct((N, OUT), jnp.float32),
    )(x, agg, w0, w1, fc1t, fc1_b, fc2t, fc2_b)


def kernel(u, edge_index, neighbor_all, emb_id, att_fc1_w, att_fc1_b,
           att_fc2_w, att_fc2_b, w, fc1_w, fc1_b, fc2_w, fc2_b):
    N, Hdim = u.shape
    E = edge_index.shape[1]
    D = neighbor_all.shape[1]
    OUT = fc2_w.shape[0]
    assert Hdim == H

    x = emb_id
    k = edge_index[0]
    i = edge_index[1]

    t32 = _pack_table_call(u, x, N)
    EH = E // 2
    ek = _edge_diffprod_kernel(EH, EH // NW, 40)
    h032a = ek(t32, k[:EH], i[:EH])
    h032b = ek(t32, k[EH:], i[EH:])

    EB = 640
    w1t_bf = att_fc1_w.T.astype(jnp.bfloat16)
    b1r = att_fc1_b.reshape(1, H)
    la = _logits_call(h032a, w1t_bf, b1r, att_fc2_w, EH, EB).reshape(EH)
    lb = _logits_call(h032b, w1t_bf, b1r, att_fc2_w, EH, EB).reshape(EH)
    logits = jnp.concatenate([la, lb])

    PAD = 8
    lg_pad = jnp.concatenate([logits, jnp.full((PAD,), -1e30, jnp.float32)])
    k_pad = jnp.concatenate([k, jnp.zeros((PAD,), jnp.int32)])
    naf = neighbor_all.reshape(N * D)
    CN = 128 // D
    agg = _neighbor_agg_kernel(N, D, CN, E)(naf, k_pad, lg_pad, x)

    NB = 1000
    return _head_call(x, agg, w[0], w[1], fc1_w.T, fc1_b.reshape(1, H),
                      fc2_w.T, fc2_b.reshape(1, OUT), N, NB, OUT)


# speedup vs baseline: 2.2916x; 1.0285x over previous
import functools

import jax
import jax.numpy as jnp
from jax import lax
from jax.experimental import pallas as pl
from jax.experimental.pallas import tpu as pltpu
from jax.experimental.pallas import tpu_sc as plsc

H = 128
NW = 32
LANES = 16


def _edge_diffprod_kernel(E, EW, CH):
    n_chunks = EW // CH
    n_pairs = n_chunks // 2
    assert n_chunks == 2 * n_pairs + 1

    mesh = plsc.VectorSubcoreMesh(core_axis_name="c", subcore_axis_name="s")

    @functools.partial(
        pl.kernel,
        out_type=jax.ShapeDtypeStruct((E, H // 2), jnp.int32),
        mesh=mesh,
        scratch_types=[
            pltpu.VMEM((EW,), jnp.int32),
            pltpu.VMEM((EW,), jnp.int32),
            pltpu.VMEM((2, CH, H), jnp.int32),
            pltpu.VMEM((2, CH, H), jnp.int32),
            pltpu.VMEM((2, CH, H // 2), jnp.int32),
            pltpu.SemaphoreType.DMA,
            pltpu.SemaphoreType.DMA,
            pltpu.SemaphoreType.DMA,
            pltpu.SemaphoreType.DMA,
            pltpu.SemaphoreType.DMA,
            pltpu.SemaphoreType.DMA,
        ],
    )
    def edge_kernel(t_hbm, k_hbm, i_hbm, h_hbm, kidx_all, iidx_all,
                    krows, irows, hbuf, sk0, sk1, si0, si1, st0, st1):
        wid = lax.axis_index("s") * 2 + lax.axis_index("c")
        base = wid * EW
        pltpu.sync_copy(k_hbm.at[pl.ds(base, EW)], kidx_all)
        pltpu.sync_copy(i_hbm.at[pl.ds(base, EW)], iidx_all)
        semk = (sk0, sk1)
        semi = (si0, si1)
        semst = (st0, st1)

        def fire(c, b):
            pltpu.async_copy(
                t_hbm.at[kidx_all.at[pl.ds(c * CH, CH)]], krows.at[b],
                semk[b])
            pltpu.async_copy(
                t_hbm.at[iidx_all.at[pl.ds(c * CH, CH)]], irows.at[b],
                semi[b])

        def wait_gather(b):
            pltpu.make_async_copy(
                t_hbm.at[kidx_all.at[pl.ds(0, CH)]], krows.at[b],
                semk[b]).wait()
            pltpu.make_async_copy(
                t_hbm.at[iidx_all.at[pl.ds(0, CH)]], irows.at[b],
                semi[b]).wait()

        MASK_HI = jnp.int32(-65536)
        MASK_LO = jnp.int32(65535)

        def _hi(wv):
            return lax.bitcast_convert_type(wv, jnp.float32)

        def _lo(wv):
            return lax.bitcast_convert_type(wv << 16, jnp.float32)

        def compute(b):
            def row_body(e, carry2):
                for l in range(H // (2 * LANES)):
                    o = l * LANES
                    ku = krows[b, e, pl.ds(o, LANES)]
                    iu = irows[b, e, pl.ds(o, LANES)]
                    kx = krows[b, e, pl.ds(H // 2 + o, LANES)]
                    ix = irows[b, e, pl.ds(H // 2 + o, LANES)]
                    h_h = lax.bitcast_convert_type(
                        (_hi(ku) - _hi(iu)) * (_hi(kx) - _hi(ix)), jnp.int32)
                    h_l = lax.bitcast_convert_type(
                        (_lo(ku) - _lo(iu)) * (_lo(kx) - _lo(ix)), jnp.int32)
                    hbuf[b, e, pl.ds(o, LANES)] = \
                        (h_h & MASK_HI) | ((h_l >> 16) & MASK_LO)
                return carry2

            lax.fori_loop(0, CH, row_body, 0)

        def fire_store(c, b):
            pltpu.async_copy(hbuf.at[b],
                             h_hbm.at[pl.ds(base + c * CH, CH)], semst[b])

        def wait_store(b):
            pltpu.make_async_copy(hbuf.at[b], h_hbm.at[pl.ds(base, CH)],
                                  semst[b]).wait()

        fire(0, 0)
        fire(1, 1)

        def pair_body(p, carry):
            c0 = 2 * p
            wait_gather(0)

            @pl.when(p > 0)
            def _():
                wait_store(0)

            compute(0)
            fire_store(c0, 0)
            fire(c0 + 2, 0)
            wait_gather(1)

            @pl.when(p > 0)
            def _():
                wait_store(1)

            compute(1)
            fire_store(c0 + 1, 1)

            @pl.when(p < n_pairs - 1)
            def _():
                fire(c0 + 3, 1)

            return carry

        lax.fori_loop(0, n_pairs, pair_body, 0)

        wait_gather(0)
        wait_store(0)
        compute(0)
        pltpu.sync_copy(hbuf.at[0],
                        h_hbm.at[pl.ds(base + (n_chunks - 1) * CH, CH)])
        wait_store(1)

    return edge_kernel


MASK16 = 65535
MASKHI = -65536


def _pack_half(v):
    r_lo = lax.bitcast_convert_type(
        v[:, 0:H // 2].astype(jnp.bfloat16).astype(jnp.float32), jnp.int32)
    r_hi = lax.bitcast_convert_type(
        v[:, H // 2:H].astype(jnp.bfloat16).astype(jnp.float32), jnp.int32)
    return ((r_lo >> 16) & MASK16) | (r_hi & MASKHI)


def _unpack_cat(w):
    f_lo = lax.bitcast_convert_type(w << 16, jnp.float32)
    f_hi = lax.bitcast_convert_type(w & MASKHI, jnp.float32)
    return jnp.concatenate([f_lo, f_hi], axis=1)


def _pack_table_call(u, x, N):
    def body(u_ref, x_ref, t_ref):
        t_ref[...] = jnp.concatenate(
            [_pack_half(u_ref[...]), _pack_half(x_ref[...])], axis=1)

    NB = 2000
    return pl.pallas_call(
        body,
        grid=(N // NB,),
        in_specs=[
            pl.BlockSpec((NB, H), lambda b: (b, 0)),
            pl.BlockSpec((NB, H), lambda b: (b, 0)),
        ],
        out_specs=pl.BlockSpec((NB, H), lambda b: (b, 0)),
        out_shape=jax.ShapeDtypeStruct((N, H), jnp.int32),
    )(u, x)


def _logits_call(h032, w1t, b1, w2, E, EB):
    def body(h_ref, w1t_ref, b1_ref, w2_ref, out_ref):
        hbf = _unpack_cat(h_ref[...]).astype(jnp.bfloat16)
        h = jnp.dot(hbf, w1t_ref[...],
                    preferred_element_type=jnp.float32)
        h = jnp.maximum(h + b1_ref[...], 0.0)
        out_ref[...] = lax.dot_general(
            w2_ref[...], h, (((1,), (1,)), ((), ())),
            preferred_element_type=jnp.float32).reshape(1, 1, EB)

    return pl.pallas_call(
        body,
        grid=(E // EB,),
        in_specs=[
            pl.BlockSpec((EB, H // 2), lambda b: (b, 0)),
            pl.BlockSpec((H, H), lambda b: (0, 0)),
            pl.BlockSpec((1, H), lambda b: (0, 0)),
            pl.BlockSpec((1, H), lambda b: (0, 0)),
        ],
        out_specs=pl.BlockSpec((1, 1, EB), lambda b: (b, 0, 0)),
        out_shape=jax.ShapeDtypeStruct((E // EB, 1, EB), jnp.float32),
    )(h032, w1t, b1, w2)


def _neighbor_agg_kernel(N, D, CN, E):
    PAIRS = CN * D
    nodes_w = (N // NW) // CN * CN
    n_chunks = nodes_w // CN
    n_pairs = n_chunks // 2
    assert n_chunks == 2 * n_pairs
    rem = N - NW * nodes_w
    assert rem <= NW
    jpre = n_chunks * PAIRS
    ES = E // (NW // 2)

    mesh = plsc.VectorSubcoreMesh(core_axis_name="c", subcore_axis_name="s")

    @functools.partial(
        pl.kernel,
        out_type=jax.ShapeDtypeStruct((N, H), jnp.float32),
        mesh=mesh,
        scratch_types=[
            pltpu.VMEM((jpre,), jnp.int32),
            pltpu.VMEM((2, PAIRS), jnp.int32),
            pltpu.VMEM((PAIRS + LANES,), jnp.float32),
            pltpu.VMEM((PAIRS + LANES,), jnp.float32),
            pltpu.VMEM((2, PAIRS, H), jnp.float32),
            pltpu.VMEM((nodes_w, H), jnp.float32),
            pltpu.VMEM((D,), jnp.int32),
            pltpu.VMEM((D,), jnp.int32),
            pltpu.VMEM((D + LANES,), jnp.float32),
            pltpu.VMEM((D, H), jnp.float32),
            pltpu.VMEM((8, H), jnp.float32),
            pltpu.VMEM((ES,), jnp.float32),
            pltpu.VMEM((LANES,), jnp.float32),
            pltpu.VMEM((LANES, LANES), jnp.float32),
            pltpu.VMEM_SHARED((2 * LANES, LANES), jnp.float32),
            pltpu.SemaphoreType.DMA,
            pltpu.SemaphoreType.DMA,
            pltpu.SemaphoreType.DMA,
            pltpu.SemaphoreType.DMA,
            pltpu.SemaphoreType.DMA,
            pltpu.SemaphoreType.DMA,
        ],
    )
    def agg_kernel(naf_hbm, kpad_hbm, lg_hbm, x_hbm, agg_hbm,
                   jidx_all, kj, av0, av1, rows, outall, ej, ekj, eav, erows,
                   eout, lgbuf, pvec, locbuf, shpart,
                   ska, skb, saa, sab, sra, srb):
        av = (av0, av1)
        wid = lax.axis_index("s") * 2 + lax.axis_index("c")
        sid = lax.axis_index("s")
        node0 = wid * nodes_w
        pltpu.sync_copy(naf_hbm.at[pl.ds(node0 * D, jpre)], jidx_all)
        semk = (ska, skb)
        sema = (saa, sab)
        semr = (sra, srb)

        pltpu.sync_copy(lg_hbm.at[pl.ds(sid * ES, ES)], lgbuf)

        def max_body(l, m):
            return jnp.maximum(m, lgbuf[pl.ds(l * LANES, LANES)])

        pm = lax.fori_loop(0, ES // LANES, max_body,
                           jnp.full((LANES,), -jnp.inf, jnp.float32))
        pvec[...] = pm
        pltpu.sync_copy(pvec, shpart.at[sid])
        plsc.subcore_barrier()
        pltpu.sync_copy(shpart.at[pl.ds(0, LANES)], locbuf)
        gm = locbuf[0, pl.ds(0, LANES)]
        for w in range(1, LANES):
            gm = jnp.maximum(gm, locbuf[w, pl.ds(0, LANES)])
        M = gm[0]
        for i in range(1, LANES):
            M = jnp.maximum(M, gm[i])

        def sum_body(l, s):
            return s + jnp.exp(lgbuf[pl.ds(l * LANES, LANES)] - M)

        ps = lax.fori_loop(0, ES // LANES, sum_body,
                           jnp.zeros((LANES,), jnp.float32))
        pvec[...] = ps
        pltpu.sync_copy(pvec, shpart.at[LANES + sid])
        plsc.subcore_barrier()
        pltpu.sync_copy(shpart.at[pl.ds(LANES, LANES)], locbuf)
        gs = locbuf[0, pl.ds(0, LANES)]
        for w in range(1, LANES):
            gs = gs + locbuf[w, pl.ds(0, LANES)]
        S = gs[0]
        for i in range(1, LANES):
            S = S + gs[i]
        rinv = jnp.ones((LANES,), jnp.float32) / jnp.full((LANES,), S,
                                                          jnp.float32)

        def xform(b):
            for j in range(PAIRS // LANES):
                sl = pl.ds(j * LANES, LANES)
                av[b][sl] = jnp.exp(av[b][sl] - M) * rinv

        def fire_kjav(c, b):
            idx = jidx_all.at[pl.ds(c * PAIRS, PAIRS)]
            pltpu.async_copy(kpad_hbm.at[idx], kj.at[b], semk[b])
            pltpu.async_copy(lg_hbm.at[idx],
                             av[b].at[pl.ds(0, PAIRS)], sema[b])

        def wait_kjav(b):
            idx = jidx_all.at[pl.ds(0, PAIRS)]
            pltpu.make_async_copy(kpad_hbm.at[idx], kj.at[b], semk[b]).wait()
            pltpu.make_async_copy(lg_hbm.at[idx],
                                  av[b].at[pl.ds(0, PAIRS)],
                                  sema[b]).wait()

        def fire_rows(b):
            pltpu.async_copy(x_hbm.at[kj.at[b]], rows.at[b], semr[b])

        def wait_rows(b):
            pltpu.make_async_copy(x_hbm.at[kj.at[b]], rows.at[b],
                                  semr[b]).wait()

        def compute(c, b):
            for n in range(CN):
                def d_body(d, acc):
                    cidx = n * D + d
                    a = av[b][pl.ds(cidx, LANES)][0]
                    return tuple(
                        acc[l] + a * rows[b, cidx, pl.ds(l * LANES, LANES)]
                        for l in range(H // LANES))

                zero = jnp.zeros((LANES,), jnp.float32)
                acc = lax.fori_loop(0, D, d_body,
                                    tuple(zero for _ in range(H // LANES)))
                row = c * CN + n
                for l in range(H // LANES):
                    outall[row, pl.ds(l * LANES, LANES)] = acc[l]

        fire_kjav(0, 0)
        wait_kjav(0)
        xform(0)
        fire_rows(0)
        fire_kjav(1, 1)

        def pair_body(p, carry):
            c0 = 2 * p
            wait_kjav(1)
            xform(1)
            fire_rows(1)
            wait_rows(0)
            compute(c0, 0)

            @pl.when(p < n_pairs - 1)
            def _():
                fire_kjav(c0 + 2, 0)

            wait_rows(1)
            compute(c0 + 1, 1)

            @pl.when(p < n_pairs - 1)
            def _():
                wait_kjav(0)
                xform(0)
                fire_rows(0)
                fire_kjav(c0 + 3, 1)

            return carry

        lax.fori_loop(0, n_pairs, pair_body, 0)
        pltpu.sync_copy(outall, agg_hbm.at[pl.ds(node0, nodes_w)])

        @pl.when(wid < rem)
        def _():
            g = NW * nodes_w + wid
            pltpu.sync_copy(naf_hbm.at[pl.ds(g * D, D)], ej)
            cpk = pltpu.async_copy(kpad_hbm.at[ej], ekj, ska)
            cpa = pltpu.async_copy(lg_hbm.at[ej], eav.at[pl.ds(0, D)], saa)
            cpk.wait()
            cpa.wait()
            for j in range(D // LANES):
                sl = pl.ds(j * LANES, LANES)
                eav[sl] = jnp.exp(eav[sl] - M) * rinv
            cpr = pltpu.async_copy(x_hbm.at[ekj], erows, sra)
            cpr.wait()

            def d_body(d, acc):
                a = eav[pl.ds(d, LANES)][0]
                return tuple(
                    acc[l] + a * erows[d, pl.ds(l * LANES, LANES)]
                    for l in range(H // LANES))

            zero = jnp.zeros((LANES,), jnp.float32)
            acc = lax.fori_loop(0, D, d_body,
                                tuple(zero for _ in range(H // LANES)))
            for l in range(H // LANES):
                eout[0, pl.ds(l * LANES, LANES)] = acc[l]
            pltpu.sync_copy(eout.at[pl.ds(0, 1)], agg_hbm.at[pl.ds(g, 1)])

    return agg_kernel


def _head_call(x, agg, w0, w1, fc1t, fc1_b, fc2t, fc2_b, N, NB, OUT):
    def body(x_ref, agg_ref, w0_ref, w1_ref, fc1t_ref, fc1b_ref, fc2t_ref,
             fc2b_ref, out_ref):
        x2 = (jnp.dot(x_ref[...], w0_ref[...],
                      preferred_element_type=jnp.float32)
              + jnp.dot(agg_ref[...], w1_ref[...],
                        preferred_element_type=jnp.float32))
        x2 = jnp.maximum(jnp.dot(x2, fc1t_ref[...],
                                 preferred_element_type=jnp.float32)
                         + fc1b_ref[...], 0.0)
        out_ref[...] = jnp.dot(x2, fc2t_ref[...],
                               preferred_element_type=jnp.float32) \
            + fc2b_ref[...]

    return pl.pallas_call(
        body,
        grid=(N // NB,),
        in_specs=[
            pl.BlockSpec((NB, H), lambda b: (b, 0)),
            pl.BlockSpec((NB, H), lambda b: (b, 0)),
            pl.BlockSpec((H, H), lambda b: (0, 0)),
            pl.BlockSpec((H, H), lambda b: (0, 0)),
            pl.BlockSpec((H, H), lambda b: (0, 0)),
            pl.BlockSpec((1, H), lambda b: (0, 0)),
            pl.BlockSpec((H, OUT), lambda b: (0, 0)),
            pl.BlockSpec((1, OUT), lambda b: (0, 0)),
        ],
        out_specs=pl.BlockSpec((NB, OUT), lambda b: (b, 0)),
        out_shape=jax.ShapeDtypeStruct((N, OUT), jnp.float32),
    )(x, agg, w0, w1, fc1t, fc1_b, fc2t, fc2_b)


def kernel(u, edge_index, neighbor_all, emb_id, att_fc1_w, att_fc1_b,
           att_fc2_w, att_fc2_b, w, fc1_w, fc1_b, fc2_w, fc2_b):
    N, Hdim = u.shape
    E = edge_index.shape[1]
    D = neighbor_all.shape[1]
    OUT = fc2_w.shape[0]
    assert Hdim == H

    x = emb_id
    k = edge_index[0]
    i = edge_index[1]

    t32 = _pack_table_call(u, x, N)
    SPLITS = ((0, 128000, 32), (128000, 256000, 32), (256000, E, 16))
    h_parts = [
        _edge_diffprod_kernel(hi - lo, (hi - lo) // NW, ch)(
            t32, k[lo:hi], i[lo:hi])
        for lo, hi, ch in SPLITS
    ]

    EB = 640
    w1t_bf = att_fc1_w.T.astype(jnp.bfloat16)
    b1r = att_fc1_b.reshape(1, H)
    logits = jnp.concatenate([
        _logits_call(hp, w1t_bf, b1r, att_fc2_w, hi - lo, EB).reshape(hi - lo)
        for hp, (lo, hi, _) in zip(h_parts, SPLITS)
    ])

    PAD = 8
    lg_pad = jnp.concatenate([logits, jnp.full((PAD,), -1e30, jnp.float32)])
    k_pad = jnp.concatenate([k, jnp.zeros((PAD,), jnp.int32)])
    naf = neighbor_all.reshape(N * D)
    CN = 128 // D
    agg = _neighbor_agg_kernel(N, D, CN, E)(naf, k_pad, lg_pad, x)

    NB = 1000
    return _head_call(x, agg, w[0], w[1], fc1_w.T, fc1_b.reshape(1, H),
                      fc2_w.T, fc2_b.reshape(1, OUT), N, NB, OUT)
